# speedup vs baseline: 1.2918x; 1.2918x over previous
import functools
import jax
import jax.numpy as jnp
from jax.experimental import pallas as pl
from jax.experimental.pallas import tpu as pltpu

_E = 8
_K = 2


def _moe_body(x_ref, wr_ref, wg_ref, wu_ref, wd_ref, o_ref):
    e = pl.program_id(1)
    x = x_ref[...]
    logits = jnp.dot(x, wr_ref[...], preferred_element_type=jnp.float32)
    m1 = jnp.max(logits, axis=-1, keepdims=True)
    idx = jax.lax.broadcasted_iota(jnp.int32, logits.shape, 1)
    big = jnp.int32(1 << 30)
    a1 = jnp.min(jnp.where(logits == m1, idx, big), axis=-1, keepdims=True)
    masked = jnp.where(idx == a1, -jnp.inf, logits)
    m2 = jnp.max(masked, axis=-1, keepdims=True)
    a2 = jnp.min(jnp.where(masked == m2, idx, big), axis=-1, keepdims=True)
    t = jnp.exp(m2 - m1)
    w1 = 1.0 / (1.0 + t)
    w2 = 1.0 - w1
    w_e = jnp.where(a1 == e, w1, 0.0) + jnp.where(a2 == e, w2, 0.0)

    g = jnp.dot(x, wg_ref[0], preferred_element_type=jnp.float32)
    u = jnp.dot(x, wu_ref[0], preferred_element_type=jnp.float32)
    h = g * jax.lax.logistic(g) * u
    y = jnp.dot(h, wd_ref[0], preferred_element_type=jnp.float32)
    contrib = w_e * y

    @pl.when(e == 0)
    def _init():
        o_ref[...] = contrib

    @pl.when(e > 0)
    def _acc():
        o_ref[...] += contrib


def kernel(x, W_router, Wg, Wu, Wd):
    T, D = x.shape
    E, _, F = Wg.shape
    BT = 256
    grid = (T // BT, E)
    return pl.pallas_call(
        _moe_body,
        grid=grid,
        in_specs=[
            pl.BlockSpec((BT, D), lambda i, e: (i, 0)),
            pl.BlockSpec((D, E), lambda i, e: (0, 0)),
            pl.BlockSpec((1, D, F), lambda i, e: (e, 0, 0)),
            pl.BlockSpec((1, D, F), lambda i, e: (e, 0, 0)),
            pl.BlockSpec((1, F, D), lambda i, e: (e, 0, 0)),
        ],
        out_specs=pl.BlockSpec((BT, D), lambda i, e: (i, 0)),
        out_shape=jax.ShapeDtypeStruct((T, D), jnp.float32),
        compiler_params=pltpu.CompilerParams(
            dimension_semantics=("parallel", "arbitrary"),
        ),
    )(x, W_router, Wg, Wu, Wd)
